# baseline (speedup 1.0000x reference)
import functools

import jax
import jax.numpy as jnp
from jax import lax
from jax.experimental import pallas as pl
from jax.experimental.pallas import tpu as pltpu
from jax.experimental.pallas import tpu_sc as plsc


def _node_tables_body(x_ref, w_ref, ei_ref, p_ref, q_ref, ei2_ref,
                      *, n, n_pad, ns, e128):
  xw = jnp.dot(x_ref[...], w_ref[...], preferred_element_type=jnp.float32)
  p_ref[:n] = xw[:, :ns]
  q_ref[:n] = xw[:, ns:]
  pad = jnp.zeros((n_pad - n, ns), jnp.float32)
  p_ref[n:] = pad
  q_ref[n:] = pad
  ei = ei_ref[...].reshape(2, e128, 128)
  ei2_ref[:e128] = ei[0]
  ei2_ref[e128:] = ei[1]


def _edge_term_body(ea_ref, wc_ref, c_ref, *, de, ns):
  be = ea_ref.shape[0]
  acc = jnp.broadcast_to(wc_ref[de:de + 1, :], (be, ns))
  for j in range(de):
    acc = acc + ea_ref[:, j:j + 1] * wc_ref[j:j + 1, :]
  fold = 128 // ns
  acc3 = acc.reshape(be // fold, fold, ns)
  c_ref[...] = jnp.concatenate([acc3[:, s, :] for s in range(fold)], axis=1)


def _finish_body(s_ref, ct_ref, w2_ref, b2_ref, o_ref):
  s = s_ref[0] + s_ref[1]
  cnt = jnp.sum(ct_ref[...], axis=1, keepdims=True)
  agg = jnp.dot(s, w2_ref[...], preferred_element_type=jnp.float32)
  agg = agg + cnt * b2_ref[...]
  o_ref[...] = agg / jnp.maximum(cnt, 1.0)


def _sc_edge_body(p_hbm, q_hbm, c_hbm, ei_hbm,
                  s_out, cnt_out,
                  sidx, didx, pg, qg, cg, msg, cnt_l, zb, shared_s,
                  isem0, isem1, esem0, esem1, gsem0, gsem1, ssem0, ssem1,
                  *, n_pad, e, chunk, chunks, ns):
  cid = lax.axis_index("c")
  sid = lax.axis_index("s")
  wid = cid * 16 + sid
  rpt = n_pad // 16
  gb = chunk // 16
  zero16 = jnp.zeros((ns,), jnp.float32)
  ones16 = jnp.ones((ns,), jnp.float32)
  isem = [isem0, isem1]
  esem = [esem0, esem1]
  gsem = [gsem0, gsem1]
  ssem = [ssem0, ssem1]
  row0 = wid * (chunk * chunks)

  def issue_idx(c):
    b, p = c % 4, c % 2
    base = row0 + c * chunk
    pltpu.async_copy(ei_hbm.at[pl.ds(base, chunk)], sidx.at[b], isem[p])
    pltpu.async_copy(ei_hbm.at[pl.ds(e + base, chunk)], didx.at[b], isem[p])
    pltpu.async_copy(c_hbm.at[pl.ds(base * ns, chunk * ns)],
                     cg.at[p], esem[p])

  def wait_idx(c):
    b, p = c % 4, c % 2
    pltpu.make_async_copy(ei_hbm.at[pl.ds(0, chunk)],
                          sidx.at[b], isem[p]).wait()
    pltpu.make_async_copy(ei_hbm.at[pl.ds(0, chunk)],
                          didx.at[b], isem[p]).wait()

  def fire_gathers(c):
    b, p = c % 4, c % 2

    @pl.loop(0, gb)
    def _(t):
      sv = sidx[b, pl.ds(t * 16, 16)]
      dv = didx[b, pl.ds(t * 16, 16)]
      pltpu.async_copy(p_hbm.at[sv], pg.at[p].at[pl.ds(t * 16, 16)], gsem[p])
      pltpu.async_copy(q_hbm.at[dv], qg.at[p].at[pl.ds(t * 16, 16)], gsem[p])

  issue_idx(0)
  issue_idx(1)
  wait_idx(0)
  fire_gathers(0)

  @pl.loop(0, rpt)
  def _(i):
    zb[i, :] = zero16
    cnt_l[i, :] = zero16

  pltpu.sync_copy(zb, shared_s.at[pl.ds(sid * rpt, rpt)])
  plsc.subcore_barrier()

  for c in range(chunks):
    p = c % 2
    b = c % 4
    if c >= 2:
      pltpu.make_async_copy(p_hbm.at[pl.ds(0, chunk)],
                            msg.at[p], ssem[p]).wait()
    if c + 1 < chunks:
      wait_idx(c + 1)
      fire_gathers(c + 1)
    pltpu.make_async_copy(p_hbm.at[pl.ds(0, chunk)], pg.at[p], gsem[p]).wait()
    pltpu.make_async_copy(q_hbm.at[pl.ds(0, chunk)], qg.at[p], gsem[p]).wait()
    pltpu.make_async_copy(c_hbm.at[pl.ds(0, chunk * ns)],
                          cg.at[p], esem[p]).wait()

    @pl.loop(0, chunk, step=4)
    def _(e0, p=p):
      for k in range(4):
        ei = e0 + k
        acc = pg[p, ei, :] + qg[p, ei, :] + cg[p, pl.ds(ei * ns, ns)]
        msg[p, ei, :] = jnp.maximum(acc, 0.0)

    @pl.loop(0, gb)
    def _(t, p=p, b=b):
      iv = didx[b, pl.ds(t * 16, 16)]
      plsc.addupdate_scatter(cnt_l, [iv >> 4, iv & 15], ones16)
      pltpu.async_copy(msg.at[p].at[pl.ds(t * 16, 16)],
                       shared_s.at[iv], ssem[p], add=True)

    if c + 2 < chunks:
      issue_idx(c + 2)

  for c in (chunks - 2, chunks - 1):
    pltpu.make_async_copy(p_hbm.at[pl.ds(0, chunk)],
                          msg.at[c % 2], ssem[c % 2]).wait()

  plsc.subcore_barrier()
  pltpu.sync_copy(shared_s.at[pl.ds(sid * rpt, rpt)],
                  s_out.at[cid].at[pl.ds(sid * rpt, rpt)])
  pltpu.sync_copy(cnt_l, cnt_out.at[wid])


def kernel(x, edge_index, edge_attr, W1, b1, W2, b2):
  n, d = x.shape
  e = edge_index.shape[1]
  de = edge_attr.shape[1]
  ns = W1.shape[1]

  nw = 32
  chunk = 400
  chunks = e // (nw * chunk)
  n_pad = ((n + 1 + 127) // 128) * 128
  e128 = e // 128

  w1ab = jnp.concatenate([W1[:d], W1[d:2 * d]], axis=1)
  w1cb = jnp.concatenate([W1[2 * d:], b1.reshape(1, ns)], axis=0)

  p_tab, q_tab, ei2 = pl.pallas_call(
      functools.partial(_node_tables_body, n=n, n_pad=n_pad, ns=ns,
                        e128=e128),
      out_shape=(
          jax.ShapeDtypeStruct((n_pad, ns), jnp.float32),
          jax.ShapeDtypeStruct((n_pad, ns), jnp.float32),
          jax.ShapeDtypeStruct((2 * e128, 128), jnp.int32),
      ),
  )(x, w1ab, edge_index)

  be = 8192
  c_tab = pl.pallas_call(
      functools.partial(_edge_term_body, de=de, ns=ns),
      grid=(e // be,),
      in_specs=[
          pl.BlockSpec((be, de), lambda i: (i, 0)),
          pl.BlockSpec((de + 1, ns), lambda i: (0, 0)),
      ],
      out_specs=pl.BlockSpec((be * ns // 128, 128), lambda i: (i, 0)),
      out_shape=jax.ShapeDtypeStruct((e * ns // 128, 128), jnp.float32),
  )(edge_attr, w1cb)

  mesh = plsc.VectorSubcoreMesh(core_axis_name="c", subcore_axis_name="s")
  sc_fn = pl.kernel(
      functools.partial(_sc_edge_body, n_pad=n_pad, e=e, chunk=chunk,
                        chunks=chunks, ns=ns),
      out_type=(
          jax.ShapeDtypeStruct((2, n_pad, ns), jnp.float32),
          jax.ShapeDtypeStruct((nw, n_pad // 16, 16), jnp.float32),
      ),
      mesh=mesh,
      compiler_params=pltpu.CompilerParams(
          needs_layout_passes=False, use_tc_tiling_on_sc=False),
      scratch_types=[
          pltpu.VMEM((4, chunk), jnp.int32),
          pltpu.VMEM((4, chunk), jnp.int32),
          pltpu.VMEM((2, chunk, ns), jnp.float32),
          pltpu.VMEM((2, chunk, ns), jnp.float32),
          pltpu.VMEM((2, chunk * ns), jnp.float32),
          pltpu.VMEM((2, chunk, ns), jnp.float32),
          pltpu.VMEM((n_pad // 16, 16), jnp.float32),
          pltpu.VMEM((n_pad // 16, ns), jnp.float32),
          pltpu.VMEM_SHARED((n_pad, ns), jnp.float32),
          pltpu.SemaphoreType.DMA,
          pltpu.SemaphoreType.DMA,
          pltpu.SemaphoreType.DMA,
          pltpu.SemaphoreType.DMA,
          pltpu.SemaphoreType.DMA,
          pltpu.SemaphoreType.DMA,
          pltpu.SemaphoreType.DMA,
          pltpu.SemaphoreType.DMA,
      ],
  )
  s_parts, cnt_parts = sc_fn(p_tab, q_tab, c_tab.reshape(-1),
                             ei2.reshape(-1))

  out = pl.pallas_call(
      _finish_body,
      out_shape=jax.ShapeDtypeStruct((n_pad, d), jnp.float32),
  )(s_parts, cnt_parts.reshape(nw, n_pad).T, W2, b2.reshape(1, d))

  return out[:n]


# speedup vs baseline: 2.8438x; 2.8438x over previous
import functools

import jax
import jax.numpy as jnp
from jax import lax
from jax.experimental import pallas as pl
from jax.experimental.pallas import tpu as pltpu
from jax.experimental.pallas import tpu_sc as plsc


def _node_tables_body(x_ref, w_ref, ei_ref, p_ref, q_ref, ei2_ref,
                      *, n, n_pad, ns, e128):
  xw = jnp.dot(x_ref[...], w_ref[...], preferred_element_type=jnp.float32)
  p_ref[:n] = xw[:, :ns]
  q_ref[:n] = xw[:, ns:]
  pad = jnp.zeros((n_pad - n, ns), jnp.float32)
  p_ref[n:] = pad
  q_ref[n:] = pad
  ei = ei_ref[...].reshape(2, e128, 128)
  ei2_ref[:e128] = ei[0]
  ei2_ref[e128:] = ei[1]


def _finish_body(s_ref, ct_ref, w2_ref, b2_ref, o_ref):
  s = s_ref[0] + s_ref[1]
  cnt = jnp.sum(ct_ref[...], axis=1, keepdims=True)
  agg = jnp.dot(s, w2_ref[...], preferred_element_type=jnp.float32)
  agg = agg + cnt * b2_ref[...]
  o_ref[...] = agg / jnp.maximum(cnt, 1.0)


def _sc_edge_body(p_hbm, q_hbm, ea_hbm, w1c_hbm, ei_hbm,
                  s_out, cnt_out,
                  sidx, didx, pg, qg, eb, msg, wcb, cnt_l, zb, shared_s,
                  isem0, isem1, esem0, esem1, gsem0, gsem1, ssem0, ssem1,
                  *, n_pad, e, chunk, chunks, ns, de):
  cid = lax.axis_index("c")
  sid = lax.axis_index("s")
  wid = cid * 16 + sid
  rpt = n_pad // 16
  gb = chunk // 16
  nblk = e // 128
  ebw = (chunk + 127) // 128 + 1
  eb_len = ebw * 128 * de
  zero16 = jnp.zeros((ns,), jnp.float32)
  ones16 = jnp.ones((ns,), jnp.float32)
  isem = [isem0, isem1]
  esem = [esem0, esem1]
  gsem = [gsem0, gsem1]
  ssem = [ssem0, ssem1]
  row0 = wid * (chunk * chunks)

  def blk0(c):
    base = row0 + c * chunk
    return jnp.minimum(base // 128, nblk - ebw)

  def issue_idx(c):
    b, p = c % 4, c % 2
    base = row0 + c * chunk
    pltpu.async_copy(ei_hbm.at[pl.ds(base, chunk)], sidx.at[b], isem[p])
    pltpu.async_copy(ei_hbm.at[pl.ds(e + base, chunk)], didx.at[b], isem[p])
    pltpu.async_copy(ea_hbm.at[pl.ds(blk0(c) * (128 * de), eb_len)],
                     eb.at[p], esem[p])

  def wait_idx(c):
    b, p = c % 4, c % 2
    pltpu.make_async_copy(ei_hbm.at[pl.ds(0, chunk)],
                          sidx.at[b], isem[p]).wait()
    pltpu.make_async_copy(ei_hbm.at[pl.ds(0, chunk)],
                          didx.at[b], isem[p]).wait()

  def fire_gathers(c):
    b, p = c % 4, c % 2

    @pl.loop(0, gb)
    def _(t):
      sv = sidx[b, pl.ds(t * 16, 16)]
      dv = didx[b, pl.ds(t * 16, 16)]
      pltpu.async_copy(p_hbm.at[sv], pg.at[p].at[pl.ds(t * 16, 16)], gsem[p])
      pltpu.async_copy(q_hbm.at[dv], qg.at[p].at[pl.ds(t * 16, 16)], gsem[p])

  pltpu.sync_copy(w1c_hbm, wcb)
  issue_idx(0)
  issue_idx(1)
  wait_idx(0)
  fire_gathers(0)

  @pl.loop(0, rpt)
  def _(i):
    zb[i, :] = zero16
    cnt_l[i, :] = zero16

  pltpu.sync_copy(zb, shared_s.at[pl.ds(sid * rpt, rpt)])
  plsc.subcore_barrier()

  wc = [wcb[j, :] for j in range(de + 1)]

  for c in range(chunks):
    p = c % 2
    b = c % 4
    if c >= 2:
      pltpu.make_async_copy(p_hbm.at[pl.ds(0, chunk)],
                            msg.at[p], ssem[p]).wait()
    if c + 1 < chunks:
      wait_idx(c + 1)
      fire_gathers(c + 1)
    pltpu.make_async_copy(p_hbm.at[pl.ds(0, chunk)], pg.at[p], gsem[p]).wait()
    pltpu.make_async_copy(q_hbm.at[pl.ds(0, chunk)], qg.at[p], gsem[p]).wait()
    pltpu.make_async_copy(ea_hbm.at[pl.ds(0, eb_len)],
                          eb.at[p], esem[p]).wait()

    boff = (row0 + c * chunk) - blk0(c) * 128

    @pl.loop(0, gb)
    def _(t, p=p):
      off = boff + t * 16
      loc = (off >> 7) * (128 * de) + (off & 127)
      va = [eb[p, pl.ds(loc + a * 128, 16)] for a in range(de)]
      for k in range(16):
        ei = t * 16 + k
        acc = pg[p, ei, :] + qg[p, ei, :] + wc[de]
        for a in range(de):
          acc = acc + va[a][k] * wc[a]
        msg[p, ei, :] = jnp.maximum(acc, 0.0)

    @pl.loop(0, gb)
    def _(t, p=p, b=b):
      iv = didx[b, pl.ds(t * 16, 16)]
      plsc.addupdate_scatter(cnt_l, [iv >> 4, iv & 15], ones16)
      pltpu.async_copy(msg.at[p].at[pl.ds(t * 16, 16)],
                       shared_s.at[iv], ssem[p], add=True)

    if c + 2 < chunks:
      issue_idx(c + 2)

  for c in (chunks - 2, chunks - 1):
    pltpu.make_async_copy(p_hbm.at[pl.ds(0, chunk)],
                          msg.at[c % 2], ssem[c % 2]).wait()

  plsc.subcore_barrier()
  pltpu.sync_copy(shared_s.at[pl.ds(sid * rpt, rpt)],
                  s_out.at[cid].at[pl.ds(sid * rpt, rpt)])
  pltpu.sync_copy(cnt_l, cnt_out.at[wid])


def kernel(x, edge_index, edge_attr, W1, b1, W2, b2):
  n, d = x.shape
  e = edge_index.shape[1]
  de = edge_attr.shape[1]
  ns = W1.shape[1]

  nw = 32
  chunk = 400
  chunks = e // (nw * chunk)
  n_pad = ((n + 1 + 127) // 128) * 128
  e128 = e // 128

  w1ab = jnp.concatenate([W1[:d], W1[d:2 * d]], axis=1)
  w1cb = jnp.concatenate([W1[2 * d:], b1.reshape(1, ns)], axis=0)

  p_tab, q_tab, ei2 = pl.pallas_call(
      functools.partial(_node_tables_body, n=n, n_pad=n_pad, ns=ns,
                        e128=e128),
      out_shape=(
          jax.ShapeDtypeStruct((n_pad, ns), jnp.float32),
          jax.ShapeDtypeStruct((n_pad, ns), jnp.float32),
          jax.ShapeDtypeStruct((2 * e128, 128), jnp.int32),
      ),
  )(x, w1ab, edge_index)

  mesh = plsc.VectorSubcoreMesh(core_axis_name="c", subcore_axis_name="s")
  sc_fn = pl.kernel(
      functools.partial(_sc_edge_body, n_pad=n_pad, e=e, chunk=chunk,
                        chunks=chunks, ns=ns, de=de),
      out_type=(
          jax.ShapeDtypeStruct((2, n_pad, ns), jnp.float32),
          jax.ShapeDtypeStruct((nw, n_pad // 16, 16), jnp.float32),
      ),
      mesh=mesh,
      compiler_params=pltpu.CompilerParams(
          needs_layout_passes=False, use_tc_tiling_on_sc=False),
      scratch_types=[
          pltpu.VMEM((4, chunk), jnp.int32),
          pltpu.VMEM((4, chunk), jnp.int32),
          pltpu.VMEM((2, chunk, ns), jnp.float32),
          pltpu.VMEM((2, chunk, ns), jnp.float32),
          pltpu.VMEM((2, ((chunk + 127) // 128 + 1) * 128 * de),
                     jnp.float32),
          pltpu.VMEM((2, chunk, ns), jnp.float32),
          pltpu.VMEM((de + 1, ns), jnp.float32),
          pltpu.VMEM((n_pad // 16, 16), jnp.float32),
          pltpu.VMEM((n_pad // 16, ns), jnp.float32),
          pltpu.VMEM_SHARED((n_pad, ns), jnp.float32),
          pltpu.SemaphoreType.DMA,
          pltpu.SemaphoreType.DMA,
          pltpu.SemaphoreType.DMA,
          pltpu.SemaphoreType.DMA,
          pltpu.SemaphoreType.DMA,
          pltpu.SemaphoreType.DMA,
          pltpu.SemaphoreType.DMA,
          pltpu.SemaphoreType.DMA,
      ],
  )
  ea_native = edge_attr.reshape(e128, 128, de).transpose(0, 2, 1).reshape(-1)
  s_parts, cnt_parts = sc_fn(p_tab, q_tab, ea_native, w1cb,
                             ei2.reshape(-1))

  out = pl.pallas_call(
      _finish_body,
      out_shape=jax.ShapeDtypeStruct((n_pad, d), jnp.float32),
  )(s_parts, cnt_parts.reshape(nw, n_pad).T, W2, b2.reshape(1, d))

  return out[:n]
